# baseline (speedup 1.0000x reference)
import jax
import jax.numpy as jnp
from jax import lax
from jax.experimental import pallas as pl
from jax.experimental.pallas import tpu as pltpu
from jax.experimental.pallas import tpu_sc as plsc

BATCH = 4096
HIST = 200
DIM = 64
LANES = 16
NSLICE = 4
W = DIM // NSLICE

NC = 2
NS = 16
NW = NC * NS
BPW = BATCH // NW
C0 = 128
C1 = HIST - C0


def _emb_mean_body(x_hbm, table_hbm, out_hbm, idx_v, out_v,
                   buf_a0, buf_a1, buf_b0, buf_b1,
                   sem_a0, sem_a1, sem_b0, sem_b1):
    wid = lax.axis_index("s") * NC + lax.axis_index("c")
    base = wid * BPW

    pltpu.sync_copy(x_hbm.at[pl.ds(base, BPW)], idx_v)

    bufs_a = (buf_a0, buf_a1)
    bufs_b = (buf_b0, buf_b1)
    sems_a = (sem_a0, sem_a1)
    sems_b = (sem_b0, sem_b1)

    def issue(b, k):
        pltpu.async_copy(table_hbm.at[idx_v.at[b, pl.ds(0, C0)]],
                         bufs_a[k], sems_a[k])
        pltpu.async_copy(table_hbm.at[idx_v.at[b, pl.ds(C0, C1)]],
                         bufs_b[k], sems_b[k])

    for k in range(2):
        issue(k, k)

    zero = jnp.zeros((LANES,), jnp.float32)

    def pair_body(l):
        for k in range(2):
            b = l + k
            pltpu.make_async_copy(table_hbm.at[idx_v.at[0, pl.ds(0, C0)]],
                                  bufs_a[k], sems_a[k]).wait()
            pltpu.make_async_copy(table_hbm.at[idx_v.at[0, pl.ds(0, C1)]],
                                  bufs_b[k], sems_b[k]).wait()

            def body_a(r, carry, _buf=bufs_a[k]):
                return tuple(c + _buf[r + i * (C0 // 4), pl.ds(0, LANES)]
                             for i, c in enumerate(carry))

            def body_b(r, carry, _buf=bufs_b[k]):
                return tuple(c + _buf[r + i * (C1 // 4), pl.ds(0, LANES)]
                             for i, c in enumerate(carry))

            acc = lax.fori_loop(0, C0 // 4, body_a, (zero,) * 4, unroll=8)
            acc = lax.fori_loop(0, C1 // 4, body_b, acc, unroll=6)

            @pl.when(b + 2 < BPW)
            def _():
                issue(b + 2, k)

            total = (acc[0] + acc[1]) + (acc[2] + acc[3])
            out_v[b, pl.ds(0, LANES)] = total * (1.0 / HIST)

    pl.loop(0, BPW, step=2)(pair_body)

    pltpu.sync_copy(out_v, out_hbm.at[pl.ds(base, BPW)])


def _emb_mean_slice(x, table_slice):
    mesh = plsc.VectorSubcoreMesh(core_axis_name="c", subcore_axis_name="s")
    return pl.kernel(
        _emb_mean_body,
        mesh=mesh,
        compiler_params=pltpu.CompilerParams(use_tc_tiling_on_sc=False),
        out_type=jax.ShapeDtypeStruct((BATCH, W), jnp.float32),
        scratch_types=[
            pltpu.VMEM((BPW, HIST), jnp.int32),
            pltpu.VMEM((BPW, W), jnp.float32),
            pltpu.VMEM((C0, W), jnp.float32),
            pltpu.VMEM((C0, W), jnp.float32),
            pltpu.VMEM((C1, W), jnp.float32),
            pltpu.VMEM((C1, W), jnp.float32),
            pltpu.SemaphoreType.DMA,
            pltpu.SemaphoreType.DMA,
            pltpu.SemaphoreType.DMA,
            pltpu.SemaphoreType.DMA,
        ],
    )(x, table_slice)


@jax.jit
def _emb_mean(x, table):
    outs = [_emb_mean_slice(x, table[:, s * W:(s + 1) * W])
            for s in range(NSLICE)]
    return jnp.concatenate(outs, axis=1)


def kernel(x, table):
    return _emb_mean(x.astype(jnp.int32), table)


# speedup vs baseline: 3.6521x; 3.6521x over previous
import jax
import jax.numpy as jnp
from jax import lax
from jax.experimental import pallas as pl
from jax.experimental.pallas import tpu as pltpu
from jax.experimental.pallas import tpu_sc as plsc

BATCH = 4096
HIST = 200
DIM = 64
LANES = 16
NCHUNK = DIM // LANES

NC = 2
NS = 16
NW = NC * NS
BPW = BATCH // NW
C0 = 128
C1 = HIST - C0


def _emb_mean_body(x_hbm, table_hbm, out_hbm, idx_v, out_v,
                   buf_a0, buf_a1, buf_a2, buf_a3,
                   buf_b0, buf_b1, buf_b2, buf_b3,
                   sem_a0, sem_a1, sem_a2, sem_a3,
                   sem_b0, sem_b1, sem_b2, sem_b3):
    wid = lax.axis_index("s") * NC + lax.axis_index("c")
    base = wid * BPW

    pltpu.sync_copy(x_hbm.at[pl.ds(base, BPW)], idx_v)

    bufs_a = (buf_a0, buf_a1, buf_a2, buf_a3)
    bufs_b = (buf_b0, buf_b1, buf_b2, buf_b3)
    sems_a = (sem_a0, sem_a1, sem_a2, sem_a3)
    sems_b = (sem_b0, sem_b1, sem_b2, sem_b3)

    def issue(b, k):
        pltpu.async_copy(table_hbm.at[idx_v.at[b, pl.ds(0, C0)]],
                         bufs_a[k], sems_a[k])
        pltpu.async_copy(table_hbm.at[idx_v.at[b, pl.ds(C0, C1)]],
                         bufs_b[k], sems_b[k])

    for k in range(4):
        issue(k, k)

    zero = jnp.zeros((LANES,), jnp.float32)

    def pair_body(l):
        for k in range(4):
            b = l + k
            pltpu.make_async_copy(table_hbm.at[idx_v.at[0, pl.ds(0, C0)]],
                                  bufs_a[k], sems_a[k]).wait()
            pltpu.make_async_copy(table_hbm.at[idx_v.at[0, pl.ds(0, C1)]],
                                  bufs_b[k], sems_b[k]).wait()

            def body_a(r, carry, _buf=bufs_a[k]):
                return tuple(c + _buf[r, pl.ds(j * LANES, LANES)]
                             for j, c in enumerate(carry))

            def body_b(r, carry, _buf=bufs_b[k]):
                return tuple(c + _buf[r, pl.ds(j * LANES, LANES)]
                             for j, c in enumerate(carry))

            acc = lax.fori_loop(0, C0, body_a, (zero,) * NCHUNK, unroll=8)
            acc = lax.fori_loop(0, C1, body_b, acc, unroll=8)

            @pl.when(b + 4 < BPW)
            def _():
                issue(b + 4, k)

            for j in range(NCHUNK):
                out_v[b, pl.ds(j * LANES, LANES)] = acc[j] * (1.0 / HIST)

    pl.loop(0, BPW, step=4)(pair_body)

    pltpu.sync_copy(out_v, out_hbm.at[pl.ds(base, BPW)])


@jax.jit
def _emb_mean(x, table):
    mesh = plsc.VectorSubcoreMesh(core_axis_name="c", subcore_axis_name="s")
    return pl.kernel(
        _emb_mean_body,
        mesh=mesh,
        compiler_params=pltpu.CompilerParams(use_tc_tiling_on_sc=False),
        out_type=jax.ShapeDtypeStruct((BATCH, DIM), jnp.float32),
        scratch_types=[
            pltpu.VMEM((BPW, HIST), jnp.int32),
            pltpu.VMEM((BPW, DIM), jnp.float32),
            pltpu.VMEM((C0, DIM), jnp.float32),
            pltpu.VMEM((C0, DIM), jnp.float32),
            pltpu.VMEM((C0, DIM), jnp.float32),
            pltpu.VMEM((C0, DIM), jnp.float32),
            pltpu.VMEM((C1, DIM), jnp.float32),
            pltpu.VMEM((C1, DIM), jnp.float32),
            pltpu.VMEM((C1, DIM), jnp.float32),
            pltpu.VMEM((C1, DIM), jnp.float32),
            pltpu.SemaphoreType.DMA,
            pltpu.SemaphoreType.DMA,
            pltpu.SemaphoreType.DMA,
            pltpu.SemaphoreType.DMA,
            pltpu.SemaphoreType.DMA,
            pltpu.SemaphoreType.DMA,
            pltpu.SemaphoreType.DMA,
            pltpu.SemaphoreType.DMA,
        ],
    )(x, table)


def kernel(x, table):
    return _emb_mean(x.astype(jnp.int32), table)
